# Initial kernel scaffold; baseline (speedup 1.0000x reference)
import functools

import jax
import jax.numpy as jnp
from jax import lax
from jax.experimental import pallas as pl
from jax.experimental.pallas import tpu as pltpu
from jax.experimental.pallas import tpu_sc as plsc

N = 10000
E = 320000
IN_CH = 128
HID = 64
NC, NS, L = 2, 16, 16
NW = NC * NS
EW = E // NW
K = 80
NCHUNK = EW // K
NP = 10240
RPT = NP // NS

_mesh = functools.partial(
    plsc.VectorSubcoreMesh,
    core_axis_name="c", subcore_axis_name="s", num_cores=NC, num_subcores=NS,
)


def _wid():
    cid = lax.axis_index("c")
    sid = lax.axis_index("s")
    return cid, sid, sid * NC + cid


def _deg_body(dst_hbm, ones_hbm, zrow_hbm, out_hbm,
              idx_v, ones_v, buf_v, acc_v, spm, sem):
    cid, sid, w = _wid()
    base = w * EW
    pltpu.sync_copy(zrow_hbm, spm.at[pl.ds(sid * RPT, RPT)])
    pltpu.sync_copy(ones_hbm, ones_v)
    plsc.subcore_barrier()

    def chunk(i, carry):
        pltpu.sync_copy(dst_hbm.at[pl.ds(base + i * K, K)], idx_v)
        pltpu.sync_copy(ones_v, spm.at[idx_v], add=True)
        return carry

    lax.fori_loop(0, NCHUNK, chunk, 0)
    plsc.subcore_barrier()

    pltpu.sync_copy(spm.at[pl.ds(sid * RPT, RPT)], buf_v)
    col0 = jnp.zeros((L,), jnp.int32)
    lane = lax.iota(jnp.int32, L)
    for j in range(RPT // L):
        vals = plsc.load_gather(buf_v, [j * L + lane, col0])
        acc_v[pl.ds(j * L, L)] = vals
    pltpu.sync_copy(acc_v, out_hbm.at[cid, pl.ds(sid * RPT, RPT)])


def _deg_call(dst):
    ones = jnp.zeros((K, L), jnp.float32).at[:, 0].set(1.0)
    zrow = jnp.zeros((RPT, L), jnp.float32)
    return pl.kernel(
        _deg_body,
        out_type=jax.ShapeDtypeStruct((NC, NP), jnp.float32),
        mesh=_mesh(),
        scratch_types=[
            pltpu.VMEM((K,), jnp.int32),
            pltpu.VMEM((K, L), jnp.float32),
            pltpu.VMEM((RPT, L), jnp.float32),
            pltpu.VMEM((RPT,), jnp.float32),
            pltpu.VMEM_SHARED((NP, L), jnp.float32),
            pltpu.SemaphoreType.DMA,
        ],
    )(dst, ones, zrow)


def _agg_body(width, src_hbm, dst_hbm, hs_hbm, zeros_hbm, out_hbm,
              idxs_v, idxd_v, rows_v, dump_v, acc_sh, sem):
    cid, sid, w = _wid()
    base = w * EW
    rpt = N // NS
    for k in range(5):
        pltpu.sync_copy(zeros_hbm, acc_sh.at[pl.ds(sid * rpt + k * 125, 125)])
    plsc.subcore_barrier()

    def chunk(i, carry):
        pltpu.sync_copy(src_hbm.at[pl.ds(base + i * K, K)], idxs_v)
        pltpu.sync_copy(dst_hbm.at[pl.ds(base + i * K, K)], idxd_v)
        pltpu.async_copy(hs_hbm.at[idxs_v], rows_v, sem).wait()
        pltpu.sync_copy(rows_v, acc_sh.at[idxd_v], add=True)
        return carry

    lax.fori_loop(0, NCHUNK, chunk, 0)
    plsc.subcore_barrier()

    for k in range(5):
        sl = pl.ds(sid * rpt + k * 125, 125)
        pltpu.sync_copy(acc_sh.at[sl], dump_v)
        pltpu.sync_copy(dump_v, out_hbm.at[cid, sl])


def _agg_call(src, dst, hs, width):
    zeros = jnp.zeros((125, width), jnp.float32)
    return pl.kernel(
        functools.partial(_agg_body, width),
        out_type=jax.ShapeDtypeStruct((NC, N, width), jnp.float32),
        mesh=_mesh(),
        scratch_types=[
            pltpu.VMEM((K,), jnp.int32),
            pltpu.VMEM((K,), jnp.int32),
            pltpu.VMEM((K, width), jnp.float32),
            pltpu.VMEM((125, width), jnp.float32),
            pltpu.VMEM_SHARED((N, width), jnp.float32),
            pltpu.SemaphoreType.DMA,
        ],
    )(src, dst, hs, zeros)


def _dec_body(src_hbm, dst_hbm, a_hbm, b_hbm, w_hbm, pb2_hbm, out_hbm,
              idxs_v, idxd_v, a_v, b_v, w_v, pb2_v, out_v, sem, sem2):
    cid, sid, w = _wid()
    base = w * EW
    pltpu.sync_copy(w_hbm, w_v)
    pltpu.sync_copy(pb2_hbm, pb2_v)
    pb2 = pb2_v[0]

    def chunk(i, carry):
        pltpu.sync_copy(src_hbm.at[pl.ds(base + i * K, K)], idxs_v)
        pltpu.sync_copy(dst_hbm.at[pl.ds(base + i * K, K)], idxd_v)
        d1 = pltpu.async_copy(a_hbm.at[idxs_v], a_v, sem)
        d2 = pltpu.async_copy(b_hbm.at[idxd_v], b_v, sem2)
        d1.wait()
        d2.wait()

        def edge(e, c2):
            acc = jnp.zeros((L,), jnp.float32)
            for q in range(HID // L):
                u = jnp.maximum(a_v[e, pl.ds(q * L, L)] + b_v[e, pl.ds(q * L, L)], 0.0)
                acc = acc + u * w_v[pl.ds(q * L, L)]
            out_v[e] = jnp.sum(acc) + pb2
            return c2

        lax.fori_loop(0, K, edge, 0)
        pltpu.sync_copy(out_v, out_hbm.at[pl.ds(base + i * K, K)])
        return carry

    lax.fori_loop(0, NCHUNK, chunk, 0)


def _dec_call(src, dst, acol, bcol, w2, pb2):
    return pl.kernel(
        _dec_body,
        out_type=jax.ShapeDtypeStruct((E,), jnp.float32),
        mesh=_mesh(),
        scratch_types=[
            pltpu.VMEM((K,), jnp.int32),
            pltpu.VMEM((K,), jnp.int32),
            pltpu.VMEM((K, HID), jnp.float32),
            pltpu.VMEM((K, HID), jnp.float32),
            pltpu.VMEM((HID,), jnp.float32),
            pltpu.VMEM((L,), jnp.float32),
            pltpu.VMEM((K,), jnp.float32),
            pltpu.SemaphoreType.DMA,
            pltpu.SemaphoreType.DMA,
        ],
    )(src, dst, acol, bcol, w2, pb2)


_BR = 2000


def _tc1_body(x_ref, w1_ref, d0_ref, d1_ref, hs_ref, dinv_ref):
    deg = d0_ref[...] + d1_ref[...] + 1.0
    dinv = lax.rsqrt(deg)
    h = jnp.dot(x_ref[...], w1_ref[...], preferred_element_type=jnp.float32)
    hs_ref[...] = h * dinv
    dinv_ref[...] = dinv


def _tc1(x, W1, deg0, deg1):
    return pl.pallas_call(
        _tc1_body,
        grid=(N // _BR,),
        in_specs=[
            pl.BlockSpec((_BR, IN_CH), lambda i: (i, 0)),
            pl.BlockSpec((IN_CH, IN_CH), lambda i: (0, 0)),
            pl.BlockSpec((_BR, 1), lambda i: (i, 0)),
            pl.BlockSpec((_BR, 1), lambda i: (i, 0)),
        ],
        out_specs=[
            pl.BlockSpec((_BR, IN_CH), lambda i: (i, 0)),
            pl.BlockSpec((_BR, 1), lambda i: (i, 0)),
        ],
        out_shape=[
            jax.ShapeDtypeStruct((N, IN_CH), jnp.float32),
            jax.ShapeDtypeStruct((N, 1), jnp.float32),
        ],
    )(x, W1, deg0, deg1)


def _tc2_body(a0_ref, a1_ref, hs1_ref, dinv_ref, b1_ref, w2_ref, hs2_ref):
    dinv = dinv_ref[...]
    h = jnp.maximum(dinv * (a0_ref[...] + a1_ref[...] + hs1_ref[...]) + b1_ref[...], 0.0)
    hs2_ref[...] = jnp.dot(h, w2_ref[...], preferred_element_type=jnp.float32) * dinv


def _tc2(a0, a1, hs1, dinv, b1, W2):
    return pl.pallas_call(
        _tc2_body,
        grid=(N // _BR,),
        in_specs=[
            pl.BlockSpec((_BR, IN_CH), lambda i: (i, 0)),
            pl.BlockSpec((_BR, IN_CH), lambda i: (i, 0)),
            pl.BlockSpec((_BR, IN_CH), lambda i: (i, 0)),
            pl.BlockSpec((_BR, 1), lambda i: (i, 0)),
            pl.BlockSpec((1, IN_CH), lambda i: (0, 0)),
            pl.BlockSpec((IN_CH, HID), lambda i: (0, 0)),
        ],
        out_specs=pl.BlockSpec((_BR, HID), lambda i: (i, 0)),
        out_shape=jax.ShapeDtypeStruct((N, HID), jnp.float32),
    )(a0, a1, hs1, dinv, b1, W2)


def _tc3_body(a0_ref, a1_ref, hs2_ref, dinv_ref, b2_ref, pt_ref, pb1_ref,
              pb_ref, acol_ref, bcol_ref):
    z = dinv_ref[...] * (a0_ref[...] + a1_ref[...] + hs2_ref[...]) + b2_ref[...]
    acol_ref[...] = jnp.dot(z, pt_ref[...], preferred_element_type=jnp.float32) + pb1_ref[...]
    bcol_ref[...] = jnp.dot(z, pb_ref[...], preferred_element_type=jnp.float32)


def _tc3(a0, a1, hs2, dinv, b2, PW1t, Pb1, PW1b):
    return pl.pallas_call(
        _tc3_body,
        grid=(N // _BR,),
        in_specs=[
            pl.BlockSpec((_BR, HID), lambda i: (i, 0)),
            pl.BlockSpec((_BR, HID), lambda i: (i, 0)),
            pl.BlockSpec((_BR, HID), lambda i: (i, 0)),
            pl.BlockSpec((_BR, 1), lambda i: (i, 0)),
            pl.BlockSpec((1, HID), lambda i: (0, 0)),
            pl.BlockSpec((HID, HID), lambda i: (0, 0)),
            pl.BlockSpec((1, HID), lambda i: (0, 0)),
            pl.BlockSpec((HID, HID), lambda i: (0, 0)),
        ],
        out_specs=[
            pl.BlockSpec((_BR, HID), lambda i: (i, 0)),
            pl.BlockSpec((_BR, HID), lambda i: (i, 0)),
        ],
        out_shape=[
            jax.ShapeDtypeStruct((N, HID), jnp.float32),
            jax.ShapeDtypeStruct((N, HID), jnp.float32),
        ],
    )(a0, a1, hs2, dinv, b2, PW1t, Pb1, PW1b)


@jax.jit
def kernel(x, edge_index, W1, b1, W2, b2, PW1, Pb1, PW2, Pb2):
    src = edge_index[0].astype(jnp.int32)
    dst = edge_index[1].astype(jnp.int32)

    degs = _deg_call(dst)
    deg0 = degs[0, :N].reshape(N, 1)
    deg1 = degs[1, :N].reshape(N, 1)

    hs1, dinv = _tc1(x, W1, deg0, deg1)
    agg1 = _agg_call(src, dst, hs1, IN_CH)
    hs2 = _tc2(agg1[0], agg1[1], hs1, dinv, b1.reshape(1, IN_CH), W2)
    agg2 = _agg_call(src, dst, hs2, HID)
    acol, bcol = _tc3(agg2[0], agg2[1], hs2, dinv, b2.reshape(1, HID),
                      PW1[:HID], Pb1.reshape(1, HID), PW1[HID:])
    out = _dec_call(src, dst, acol, bcol, PW2.reshape(HID),
                    jnp.broadcast_to(Pb2, (L,)))
    return out


# speedup vs baseline: 9.6480x; 9.6480x over previous
import functools

import jax
import jax.numpy as jnp
from jax import lax
from jax.experimental import pallas as pl
from jax.experimental.pallas import tpu as pltpu
from jax.experimental.pallas import tpu_sc as plsc

N = 10000
E = 320000
IN_CH = 128
HID = 64
NC, NS, L = 2, 16, 16
NW = NC * NS
EW = E // NW
K = 80
NCHUNK = EW // K
NP = 10240
RPT = NP // NS

_mesh = functools.partial(
    plsc.VectorSubcoreMesh,
    core_axis_name="c", subcore_axis_name="s", num_cores=NC, num_subcores=NS,
)
_SC_PARAMS = pltpu.CompilerParams(needs_layout_passes=False)


def _wid():
    cid = lax.axis_index("c")
    sid = lax.axis_index("s")
    return cid, sid, sid * NC + cid


def _deg_body(dst_hbm, out_hbm, idx_v, deg_v, buf_v, acc_v, spm, sem):
    cid, sid, w = _wid()
    base = w * EW
    ones = jnp.ones((L,), jnp.float32)

    def zero(j, carry):
        deg_v[pl.ds(j * L, L)] = jnp.zeros((L,), jnp.float32)
        return carry

    lax.fori_loop(0, NP // L, zero, 0)

    def chunk(i, carry):
        pltpu.sync_copy(dst_hbm.at[pl.ds(base + i * K, K)], idx_v)

        def scat(j, c2):
            plsc.addupdate_scatter(deg_v, [idx_v[pl.ds(j * L, L)]], ones)
            return c2

        lax.fori_loop(0, K // L, scat, 0)
        return carry

    lax.fori_loop(0, NCHUNK, chunk, 0)

    pltpu.sync_copy(deg_v, spm.at[pl.ds(sid * NP, NP)])
    plsc.subcore_barrier()
    lax.fori_loop(0, RPT // L, zero, 0)

    for r in range(NS):
        pltpu.sync_copy(spm.at[pl.ds(r * NP + sid * RPT, RPT)], buf_v)

        def add(j, carry):
            sl = pl.ds(j * L, L)
            deg_v[sl] = deg_v[sl] + buf_v[sl]
            return carry

        lax.fori_loop(0, RPT // L, add, 0)

    pltpu.sync_copy(deg_v.at[pl.ds(0, RPT)],
                    out_hbm.at[pl.ds(cid * NP + sid * RPT, RPT)])


def _deg_call(dst):
    return pl.kernel(
        _deg_body,
        out_type=jax.ShapeDtypeStruct((NC * NP,), jnp.float32),
        mesh=_mesh(),
        compiler_params=_SC_PARAMS,
        scratch_types=[
            pltpu.VMEM((K,), jnp.int32),
            pltpu.VMEM((NP,), jnp.float32),
            pltpu.VMEM((RPT,), jnp.float32),
            pltpu.VMEM((RPT,), jnp.float32),
            pltpu.VMEM_SHARED((NS * NP,), jnp.float32),
            pltpu.SemaphoreType.DMA,
        ],
    )(dst)


def _agg_body(width, src_hbm, dst_hbm, hs_hbm, zeros_hbm, out_hbm,
              idxs_v, idxd_v, rows_v, dump_v, acc_sh, sem):
    cid, sid, w = _wid()
    base = w * EW
    rpt = NP // NS
    for k in range(5):
        pltpu.sync_copy(zeros_hbm, acc_sh.at[pl.ds(sid * rpt + k * 128, 128)])
    plsc.subcore_barrier()

    def chunk(i, carry):
        pltpu.sync_copy(src_hbm.at[pl.ds(base + i * K, K)], idxs_v)
        pltpu.sync_copy(dst_hbm.at[pl.ds(base + i * K, K)], idxd_v)
        pltpu.async_copy(hs_hbm.at[idxs_v], rows_v, sem).wait()
        pltpu.sync_copy(rows_v, acc_sh.at[idxd_v], add=True)
        return carry

    lax.fori_loop(0, NCHUNK, chunk, 0)
    plsc.subcore_barrier()

    for k in range(5):
        sl = pl.ds(sid * rpt + k * 128, 128)
        pltpu.sync_copy(acc_sh.at[sl], dump_v)
        pltpu.sync_copy(dump_v, out_hbm.at[cid, sl])


def _agg_call(src, dst, hs, width):
    zeros = jnp.zeros((128, width), jnp.float32)
    return pl.kernel(
        functools.partial(_agg_body, width),
        out_type=jax.ShapeDtypeStruct((NC, NP, width), jnp.float32),
        mesh=_mesh(),
        compiler_params=_SC_PARAMS,
        scratch_types=[
            pltpu.VMEM((K,), jnp.int32),
            pltpu.VMEM((K,), jnp.int32),
            pltpu.VMEM((K, width), jnp.float32),
            pltpu.VMEM((128, width), jnp.float32),
            pltpu.VMEM_SHARED((NP, width), jnp.float32),
            pltpu.SemaphoreType.DMA,
        ],
    )(src, dst, hs, zeros)


def _dec_body(src_hbm, dst_hbm, c_hbm, w_hbm, pb2_hbm, out_hbm,
              idxs_v, idxd_v, a_v, b_v, w_v, pb2_v, out_v, sem, sem2):
    cid, sid, w = _wid()
    base = w * EW
    pltpu.sync_copy(w_hbm, w_v)
    pltpu.sync_copy(pb2_hbm, pb2_v)
    pb2vec = pb2_v[...]
    lane = lax.iota(jnp.int32, L)

    def chunk(i, carry):
        pltpu.sync_copy(src_hbm.at[pl.ds(base + i * K, K)], idxs_v)
        pltpu.sync_copy(dst_hbm.at[pl.ds(base + i * K, K)], idxd_v)
        d1 = pltpu.async_copy(c_hbm.at[idxs_v], a_v, sem)
        d2 = pltpu.async_copy(c_hbm.at[idxd_v], b_v, sem2)
        d1.wait()
        d2.wait()

        def grp(g, c2):
            res = jnp.zeros((L,), jnp.float32)
            for t in range(L):
                e = g * L + t
                acc = jnp.zeros((L,), jnp.float32)
                for q in range(HID // L):
                    u = jnp.maximum(
                        a_v[e, pl.ds(q * L, L)]
                        + b_v[e, pl.ds(HID + q * L, L)], 0.0)
                    acc = acc + u * w_v[pl.ds(q * L, L)]
                res = jnp.where(lane == t, jnp.sum(acc), res)
            out_v[pl.ds(g * L, L)] = res + pb2vec
            return c2

        lax.fori_loop(0, K // L, grp, 0)
        pltpu.sync_copy(out_v, out_hbm.at[pl.ds(base + i * K, K)])
        return carry

    lax.fori_loop(0, NCHUNK, chunk, 0)


def _dec_call(src, dst, ctab, w2, pb2):
    return pl.kernel(
        _dec_body,
        out_type=jax.ShapeDtypeStruct((E,), jnp.float32),
        mesh=_mesh(),
        compiler_params=_SC_PARAMS,
        scratch_types=[
            pltpu.VMEM((K,), jnp.int32),
            pltpu.VMEM((K,), jnp.int32),
            pltpu.VMEM((K, IN_CH), jnp.float32),
            pltpu.VMEM((K, IN_CH), jnp.float32),
            pltpu.VMEM((HID,), jnp.float32),
            pltpu.VMEM((L,), jnp.float32),
            pltpu.VMEM((K,), jnp.float32),
            pltpu.SemaphoreType.DMA,
            pltpu.SemaphoreType.DMA,
        ],
    )(src, dst, ctab, w2, pb2)


_BR = 2000


def _tc1_body(x_ref, w1_ref, d0_ref, d1_ref, hs_ref, dinv_ref):
    deg = d0_ref[...] + d1_ref[...] + 1.0
    dinv = lax.rsqrt(deg)
    h = jnp.dot(x_ref[...], w1_ref[...], preferred_element_type=jnp.float32)
    hs_ref[...] = h * dinv
    dinv_ref[...] = dinv


def _tc1(x, W1, deg0, deg1):
    return pl.pallas_call(
        _tc1_body,
        grid=(N // _BR,),
        in_specs=[
            pl.BlockSpec((_BR, IN_CH), lambda i: (i, 0)),
            pl.BlockSpec((IN_CH, IN_CH), lambda i: (0, 0)),
            pl.BlockSpec((_BR, 1), lambda i: (i, 0)),
            pl.BlockSpec((_BR, 1), lambda i: (i, 0)),
        ],
        out_specs=[
            pl.BlockSpec((_BR, IN_CH), lambda i: (i, 0)),
            pl.BlockSpec((_BR, 1), lambda i: (i, 0)),
        ],
        out_shape=[
            jax.ShapeDtypeStruct((N, IN_CH), jnp.float32),
            jax.ShapeDtypeStruct((N, 1), jnp.float32),
        ],
    )(x, W1, deg0, deg1)


def _tc2_body(a0_ref, a1_ref, hs1_ref, dinv_ref, b1_ref, w2_ref, pw1r_ref,
              m_ref):
    dinv = dinv_ref[...]
    h = jnp.maximum(dinv * (a0_ref[...] + a1_ref[...] + hs1_ref[...]) + b1_ref[...], 0.0)
    hs2 = jnp.dot(h, w2_ref[...], preferred_element_type=jnp.float32) * dinv
    m_ref[...] = jnp.dot(hs2, pw1r_ref[...], preferred_element_type=jnp.float32)


def _tc2(a0, a1, hs1, dinv, b1, W2, PW1r):
    return pl.pallas_call(
        _tc2_body,
        grid=(N // _BR,),
        in_specs=[
            pl.BlockSpec((_BR, IN_CH), lambda i: (i, 0)),
            pl.BlockSpec((_BR, IN_CH), lambda i: (i, 0)),
            pl.BlockSpec((_BR, IN_CH), lambda i: (i, 0)),
            pl.BlockSpec((_BR, 1), lambda i: (i, 0)),
            pl.BlockSpec((1, IN_CH), lambda i: (0, 0)),
            pl.BlockSpec((IN_CH, HID), lambda i: (0, 0)),
            pl.BlockSpec((HID, IN_CH), lambda i: (0, 0)),
        ],
        out_specs=pl.BlockSpec((_BR, IN_CH), lambda i: (i, 0)),
        out_shape=jax.ShapeDtypeStruct((N, IN_CH), jnp.float32),
    )(a0, a1, hs1, dinv, b1, W2, PW1r)


def _tc3_body(s0_ref, s1_ref, m_ref, dinv_ref, b2_ref, pw1r_ref, pb1e_ref,
              c_ref):
    cab = jnp.dot(b2_ref[...], pw1r_ref[...],
                  preferred_element_type=jnp.float32) + pb1e_ref[...]
    c_ref[...] = dinv_ref[...] * (s0_ref[...] + s1_ref[...] + m_ref[...]) + cab


def _tc3(s0, s1, m, dinv, b2, PW1r, Pb1e):
    return pl.pallas_call(
        _tc3_body,
        grid=(N // _BR,),
        in_specs=[
            pl.BlockSpec((_BR, IN_CH), lambda i: (i, 0)),
            pl.BlockSpec((_BR, IN_CH), lambda i: (i, 0)),
            pl.BlockSpec((_BR, IN_CH), lambda i: (i, 0)),
            pl.BlockSpec((_BR, 1), lambda i: (i, 0)),
            pl.BlockSpec((1, HID), lambda i: (0, 0)),
            pl.BlockSpec((HID, IN_CH), lambda i: (0, 0)),
            pl.BlockSpec((1, IN_CH), lambda i: (0, 0)),
        ],
        out_specs=pl.BlockSpec((_BR, IN_CH), lambda i: (i, 0)),
        out_shape=jax.ShapeDtypeStruct((N, IN_CH), jnp.float32),
    )(s0, s1, m, dinv, b2, PW1r, Pb1e)


@jax.jit
def kernel(x, edge_index, W1, b1, W2, b2, PW1, Pb1, PW2, Pb2):
    src = edge_index[0].astype(jnp.int32)
    dst = edge_index[1].astype(jnp.int32)

    degs = _deg_call(dst)
    deg0 = degs[:N].reshape(N, 1)
    deg1 = degs[NP:NP + N].reshape(N, 1)

    PW1r = PW1.reshape(2, HID, HID).transpose(1, 0, 2).reshape(HID, 2 * HID)
    Pb1e = jnp.concatenate([Pb1, jnp.zeros_like(Pb1)]).reshape(1, 2 * HID)

    hs1, dinv = _tc1(x, W1, deg0, deg1)
    agg1 = _agg_call(src, dst, hs1, IN_CH)
    m = _tc2(agg1[0, :N], agg1[1, :N], hs1, dinv, b1.reshape(1, IN_CH), W2,
             PW1r)
    aggm = _agg_call(src, dst, m, IN_CH)
    ctab = _tc3(aggm[0, :N], aggm[1, :N], m, dinv, b2.reshape(1, HID),
                PW1r, Pb1e)
    out = _dec_call(src, dst, ctab, PW2.reshape(HID),
                    jnp.broadcast_to(Pb2, (L,)))
    return out


# speedup vs baseline: 19.8496x; 2.0574x over previous
import functools

import jax
import jax.numpy as jnp
from jax import lax
from jax.experimental import pallas as pl
from jax.experimental.pallas import tpu as pltpu
from jax.experimental.pallas import tpu_sc as plsc

N = 10000
E = 320000
IN_CH = 128
HID = 64
NC, NS, L = 2, 16, 16
NW = NC * NS
EW = E // NW
K = 80
NCHUNK = EW // K
NP = 10240
RPT = NP // NS

_mesh = functools.partial(
    plsc.VectorSubcoreMesh,
    core_axis_name="c", subcore_axis_name="s", num_cores=NC, num_subcores=NS,
)
_SC_PARAMS = pltpu.CompilerParams(needs_layout_passes=False)


def _wid():
    cid = lax.axis_index("c")
    sid = lax.axis_index("s")
    return cid, sid, sid * NC + cid


def _deg_body(dst_hbm, out_hbm, idx_v, deg_v, buf_v, acc_v, spm, sem):
    cid, sid, w = _wid()
    base = w * EW
    ones = jnp.ones((L,), jnp.float32)

    def zero(j, carry):
        deg_v[pl.ds(j * L, L)] = jnp.zeros((L,), jnp.float32)
        return carry

    lax.fori_loop(0, NP // L, zero, 0)

    def chunk(i, carry):
        pltpu.sync_copy(dst_hbm.at[pl.ds(base + i * K, K)], idx_v)

        def scat(j, c2):
            plsc.addupdate_scatter(deg_v, [idx_v[pl.ds(j * L, L)]], ones)
            return c2

        lax.fori_loop(0, K // L, scat, 0)
        return carry

    lax.fori_loop(0, NCHUNK, chunk, 0)

    pltpu.sync_copy(deg_v, spm.at[pl.ds(sid * NP, NP)])
    plsc.subcore_barrier()
    lax.fori_loop(0, RPT // L, zero, 0)

    for r in range(NS):
        pltpu.sync_copy(spm.at[pl.ds(r * NP + sid * RPT, RPT)], buf_v)

        def add(j, carry):
            sl = pl.ds(j * L, L)
            deg_v[sl] = deg_v[sl] + buf_v[sl]
            return carry

        lax.fori_loop(0, RPT // L, add, 0)

    pltpu.sync_copy(deg_v.at[pl.ds(0, RPT)],
                    out_hbm.at[pl.ds(cid * NP + sid * RPT, RPT)])


def _deg_call(dst):
    return pl.kernel(
        _deg_body,
        out_type=jax.ShapeDtypeStruct((NC * NP,), jnp.float32),
        mesh=_mesh(),
        compiler_params=_SC_PARAMS,
        scratch_types=[
            pltpu.VMEM((K,), jnp.int32),
            pltpu.VMEM((NP,), jnp.float32),
            pltpu.VMEM((RPT,), jnp.float32),
            pltpu.VMEM((RPT,), jnp.float32),
            pltpu.VMEM_SHARED((NS * NP,), jnp.float32),
            pltpu.SemaphoreType.DMA,
        ],
    )(dst)


def _agg_body(width, src_hbm, dst_hbm, hs_hbm, zeros_hbm, out_hbm,
              idxs_all, idxd_all, idxd_v, rows0, rows1, acc_sh,
              sem0, sem1):
    cid, sid, w = _wid()
    base = w * EW
    rpt = NP // NS
    for k in range(5):
        pltpu.sync_copy(zeros_hbm, acc_sh.at[pl.ds(sid * rpt + k * 128, 128)])
    pltpu.sync_copy(src_hbm.at[pl.ds(base, EW)], idxs_all)
    pltpu.sync_copy(dst_hbm.at[pl.ds(base, EW)], idxd_all)
    plsc.subcore_barrier()

    rows = (rows0, rows1)
    sems = (sem0, sem1)

    def fire(i, b):
        pltpu.async_copy(hs_hbm.at[idxs_all.at[pl.ds(i * K, K)]],
                         rows[b], sems[b])

    def finish(i, b):
        pltpu.make_async_copy(hs_hbm.at[pl.ds(0, K)], rows[b], sems[b]).wait()
        for j in range(K // L):
            idxd_v[pl.ds(j * L, L)] = idxd_all[pl.ds(i * K + j * L, L)]
        pltpu.sync_copy(rows[b], acc_sh.at[idxd_v], add=True)

    for b in range(2):
        fire(b, b)

    def pair(i2, carry):
        for b in range(2):
            i = i2 * 2 + b
            finish(i, b)

            @pl.when(i + 2 < NCHUNK)
            def _():
                fire(i + 2, b)
        return carry

    lax.fori_loop(0, NCHUNK // 2, pair, 0)
    finish(NCHUNK - 1, (NCHUNK - 1) % 2)
    plsc.subcore_barrier()

    for k in range(8):
        sl = pl.ds(sid * rpt + k * K, K)
        pltpu.sync_copy(acc_sh.at[sl], rows0)
        pltpu.sync_copy(rows0, out_hbm.at[cid, sl])


def _agg_call(src, dst, hs, width):
    zeros = jnp.zeros((128, width), jnp.float32)
    return pl.kernel(
        functools.partial(_agg_body, width),
        out_type=jax.ShapeDtypeStruct((NC, NP, width), jnp.float32),
        mesh=_mesh(),
        compiler_params=_SC_PARAMS,
        scratch_types=[
            pltpu.VMEM((EW,), jnp.int32),
            pltpu.VMEM((EW,), jnp.int32),
            pltpu.VMEM((K,), jnp.int32),
            pltpu.VMEM((K, width), jnp.float32),
            pltpu.VMEM((K, width), jnp.float32),
            pltpu.VMEM_SHARED((NP, width), jnp.float32),
            pltpu.SemaphoreType.DMA,
            pltpu.SemaphoreType.DMA,
        ],
    )(src, dst, hs, zeros)


def _dec_body(src_hbm, dst_hbm, c_hbm, w_hbm, pb2_hbm, out_hbm,
              idxs_all, idxd_all, a0, a1, b0, b1, w_v, pb2_v, out_v,
              sa0, sa1, sb0, sb1):
    cid, sid, w = _wid()
    base = w * EW
    pltpu.sync_copy(src_hbm.at[pl.ds(base, EW)], idxs_all)
    pltpu.sync_copy(dst_hbm.at[pl.ds(base, EW)], idxd_all)
    pltpu.sync_copy(w_hbm, w_v)
    pltpu.sync_copy(pb2_hbm, pb2_v)
    pb2vec = pb2_v[...]
    lane = lax.iota(jnp.int32, L)
    abufs = (a0, a1)
    bbufs = (b0, b1)
    sas = (sa0, sa1)
    sbs = (sb0, sb1)

    def fire(i, b):
        pltpu.async_copy(c_hbm.at[idxs_all.at[pl.ds(i * K, K)]],
                         abufs[b], sas[b])
        pltpu.async_copy(c_hbm.at[idxd_all.at[pl.ds(i * K, K)]],
                         bbufs[b], sbs[b])

    def finish(i, b):
        a_v, b_v = abufs[b], bbufs[b]
        pltpu.make_async_copy(c_hbm.at[pl.ds(0, K)], a_v, sas[b]).wait()
        pltpu.make_async_copy(c_hbm.at[pl.ds(0, K)], b_v, sbs[b]).wait()

        def grp(g, c2):
            res = jnp.zeros((L,), jnp.float32)
            for t in range(L):
                e = g * L + t
                acc = jnp.zeros((L,), jnp.float32)
                for q in range(HID // L):
                    u = jnp.maximum(
                        a_v[e, pl.ds(q * L, L)]
                        + b_v[e, pl.ds(HID + q * L, L)], 0.0)
                    acc = acc + u * w_v[pl.ds(q * L, L)]
                res = jnp.where(lane == t, jnp.sum(acc), res)
            out_v[pl.ds(g * L, L)] = res + pb2vec
            return c2

        lax.fori_loop(0, K // L, grp, 0)
        pltpu.sync_copy(out_v, out_hbm.at[pl.ds(base + i * K, K)])

    for b in range(2):
        fire(b, b)

    def pair(i2, carry):
        for b in range(2):
            i = i2 * 2 + b
            finish(i, b)

            @pl.when(i + 2 < NCHUNK)
            def _():
                fire(i + 2, b)
        return carry

    lax.fori_loop(0, NCHUNK // 2, pair, 0)
    finish(NCHUNK - 1, (NCHUNK - 1) % 2)


def _dec_call(src, dst, ctab, w2, pb2):
    return pl.kernel(
        _dec_body,
        out_type=jax.ShapeDtypeStruct((E,), jnp.float32),
        mesh=_mesh(),
        compiler_params=_SC_PARAMS,
        scratch_types=[
            pltpu.VMEM((EW,), jnp.int32),
            pltpu.VMEM((EW,), jnp.int32),
            pltpu.VMEM((K, IN_CH), jnp.float32),
            pltpu.VMEM((K, IN_CH), jnp.float32),
            pltpu.VMEM((K, IN_CH), jnp.float32),
            pltpu.VMEM((K, IN_CH), jnp.float32),
            pltpu.VMEM((HID,), jnp.float32),
            pltpu.VMEM((L,), jnp.float32),
            pltpu.VMEM((K,), jnp.float32),
            pltpu.SemaphoreType.DMA,
            pltpu.SemaphoreType.DMA,
            pltpu.SemaphoreType.DMA,
            pltpu.SemaphoreType.DMA,
        ],
    )(src, dst, ctab, w2, pb2)


_BR = 2000


def _tc1_body(x_ref, w1_ref, d0_ref, d1_ref, hs_ref, dinv_ref):
    deg = d0_ref[...] + d1_ref[...] + 1.0
    dinv = lax.rsqrt(deg)
    h = jnp.dot(x_ref[...], w1_ref[...], preferred_element_type=jnp.float32)
    hs_ref[...] = h * dinv
    dinv_ref[...] = dinv


def _tc1(x, W1, deg0, deg1):
    return pl.pallas_call(
        _tc1_body,
        grid=(N // _BR,),
        in_specs=[
            pl.BlockSpec((_BR, IN_CH), lambda i: (i, 0)),
            pl.BlockSpec((IN_CH, IN_CH), lambda i: (0, 0)),
            pl.BlockSpec((_BR, 1), lambda i: (i, 0)),
            pl.BlockSpec((_BR, 1), lambda i: (i, 0)),
        ],
        out_specs=[
            pl.BlockSpec((_BR, IN_CH), lambda i: (i, 0)),
            pl.BlockSpec((_BR, 1), lambda i: (i, 0)),
        ],
        out_shape=[
            jax.ShapeDtypeStruct((N, IN_CH), jnp.float32),
            jax.ShapeDtypeStruct((N, 1), jnp.float32),
        ],
    )(x, W1, deg0, deg1)


def _tc2_body(a0_ref, a1_ref, hs1_ref, dinv_ref, b1_ref, w2_ref, pw1r_ref,
              m_ref):
    dinv = dinv_ref[...]
    h = jnp.maximum(dinv * (a0_ref[...] + a1_ref[...] + hs1_ref[...]) + b1_ref[...], 0.0)
    hs2 = jnp.dot(h, w2_ref[...], preferred_element_type=jnp.float32) * dinv
    m_ref[...] = jnp.dot(hs2, pw1r_ref[...], preferred_element_type=jnp.float32)


def _tc2(a0, a1, hs1, dinv, b1, W2, PW1r):
    return pl.pallas_call(
        _tc2_body,
        grid=(N // _BR,),
        in_specs=[
            pl.BlockSpec((_BR, IN_CH), lambda i: (i, 0)),
            pl.BlockSpec((_BR, IN_CH), lambda i: (i, 0)),
            pl.BlockSpec((_BR, IN_CH), lambda i: (i, 0)),
            pl.BlockSpec((_BR, 1), lambda i: (i, 0)),
            pl.BlockSpec((1, IN_CH), lambda i: (0, 0)),
            pl.BlockSpec((IN_CH, HID), lambda i: (0, 0)),
            pl.BlockSpec((HID, IN_CH), lambda i: (0, 0)),
        ],
        out_specs=pl.BlockSpec((_BR, IN_CH), lambda i: (i, 0)),
        out_shape=jax.ShapeDtypeStruct((N, IN_CH), jnp.float32),
    )(a0, a1, hs1, dinv, b1, W2, PW1r)


def _tc3_body(s0_ref, s1_ref, m_ref, dinv_ref, b2_ref, pw1r_ref, pb1e_ref,
              c_ref):
    cab = jnp.dot(b2_ref[...], pw1r_ref[...],
                  preferred_element_type=jnp.float32) + pb1e_ref[...]
    c_ref[...] = dinv_ref[...] * (s0_ref[...] + s1_ref[...] + m_ref[...]) + cab


def _tc3(s0, s1, m, dinv, b2, PW1r, Pb1e):
    return pl.pallas_call(
        _tc3_body,
        grid=(N // _BR,),
        in_specs=[
            pl.BlockSpec((_BR, IN_CH), lambda i: (i, 0)),
            pl.BlockSpec((_BR, IN_CH), lambda i: (i, 0)),
            pl.BlockSpec((_BR, IN_CH), lambda i: (i, 0)),
            pl.BlockSpec((_BR, 1), lambda i: (i, 0)),
            pl.BlockSpec((1, HID), lambda i: (0, 0)),
            pl.BlockSpec((HID, IN_CH), lambda i: (0, 0)),
            pl.BlockSpec((1, IN_CH), lambda i: (0, 0)),
        ],
        out_specs=pl.BlockSpec((_BR, IN_CH), lambda i: (i, 0)),
        out_shape=jax.ShapeDtypeStruct((N, IN_CH), jnp.float32),
    )(s0, s1, m, dinv, b2, PW1r, Pb1e)


@jax.jit
def kernel(x, edge_index, W1, b1, W2, b2, PW1, Pb1, PW2, Pb2):
    src = edge_index[0].astype(jnp.int32)
    dst = edge_index[1].astype(jnp.int32)

    degs = _deg_call(dst)
    deg0 = degs[:N].reshape(N, 1)
    deg1 = degs[NP:NP + N].reshape(N, 1)

    PW1r = PW1.reshape(2, HID, HID).transpose(1, 0, 2).reshape(HID, 2 * HID)
    Pb1e = jnp.concatenate([Pb1, jnp.zeros_like(Pb1)]).reshape(1, 2 * HID)

    hs1, dinv = _tc1(x, W1, deg0, deg1)
    agg1 = _agg_call(src, dst, hs1, IN_CH)
    m = _tc2(agg1[0, :N], agg1[1, :N], hs1, dinv, b1.reshape(1, IN_CH), W2,
             PW1r)
    aggm = _agg_call(src, dst, m, IN_CH)
    ctab = _tc3(aggm[0, :N], aggm[1, :N], m, dinv, b2.reshape(1, HID),
                PW1r, Pb1e)
    out = _dec_call(src, dst, ctab, PW2.reshape(HID),
                    jnp.broadcast_to(Pb2, (L,)))
    return out


# speedup vs baseline: 23.3445x; 1.1761x over previous
import functools

import jax
import jax.numpy as jnp
from jax import lax
from jax.experimental import pallas as pl
from jax.experimental.pallas import tpu as pltpu
from jax.experimental.pallas import tpu_sc as plsc

N = 10000
E = 320000
IN_CH = 128
HID = 64
NC, NS, L = 2, 16, 16
NW = NC * NS
EW = E // NW
K = 80
KB = 128
NFULL = EW // KB
KTAIL = EW - NFULL * KB
NP = 10240
RPT = NP // NS

_mesh = functools.partial(
    plsc.VectorSubcoreMesh,
    core_axis_name="c", subcore_axis_name="s", num_cores=NC, num_subcores=NS,
)
_SC_PARAMS = pltpu.CompilerParams(needs_layout_passes=False)


def _wid():
    cid = lax.axis_index("c")
    sid = lax.axis_index("s")
    return cid, sid, sid * NC + cid


def _deg_body(dst_hbm, out_hbm, idx_all, deg_v, buf_v, spm, sem):
    cid, sid, w = _wid()
    base = w * EW
    ones = jnp.ones((L,), jnp.float32)

    def zero(j, carry):
        deg_v[pl.ds(j * L, L)] = jnp.zeros((L,), jnp.float32)
        return carry

    lax.fori_loop(0, NP // L, zero, 0)
    pltpu.sync_copy(dst_hbm.at[pl.ds(base, EW)], idx_all)

    def scat(j, c2):
        plsc.addupdate_scatter(deg_v, [idx_all[pl.ds(j * L, L)]], ones)
        return c2

    lax.fori_loop(0, EW // L, scat, 0)

    pltpu.sync_copy(deg_v, spm.at[pl.ds(sid * NP, NP)])
    plsc.subcore_barrier()
    lax.fori_loop(0, RPT // L, zero, 0)

    for r in range(NS):
        pltpu.sync_copy(spm.at[pl.ds(r * NP + sid * RPT, RPT)], buf_v)

        def add(j, carry):
            sl = pl.ds(j * L, L)
            deg_v[sl] = deg_v[sl] + buf_v[sl]
            return carry

        lax.fori_loop(0, RPT // L, add, 0)

    pltpu.sync_copy(deg_v.at[pl.ds(0, RPT)],
                    out_hbm.at[pl.ds(cid * NP + sid * RPT, RPT)])


def _deg_call(dst):
    return pl.kernel(
        _deg_body,
        out_type=jax.ShapeDtypeStruct((NC * NP,), jnp.float32),
        mesh=_mesh(),
        compiler_params=_SC_PARAMS,
        scratch_types=[
            pltpu.VMEM((EW,), jnp.int32),
            pltpu.VMEM((NP,), jnp.float32),
            pltpu.VMEM((RPT,), jnp.float32),
            pltpu.VMEM_SHARED((NS * NP,), jnp.float32),
            pltpu.SemaphoreType.DMA,
        ],
    )(dst)


def _agg_body(width, src_hbm, dst_hbm, hs_hbm, zeros_hbm, out_hbm,
              idxs_all, idxd0, idxd1, idxt, rows0, rows1, acc_sh,
              sg0, sg1, si0, si1):
    cid, sid, w = _wid()
    base = w * EW
    rpt = NP // NS
    for k in range(5):
        pltpu.sync_copy(zeros_hbm, acc_sh.at[pl.ds(sid * rpt + k * 128, 128)])
    pltpu.sync_copy(src_hbm.at[pl.ds(base, EW)], idxs_all)
    plsc.subcore_barrier()

    rows = (rows0, rows1)
    idxd = (idxd0, idxd1)
    sg = (sg0, sg1)
    si = (si0, si1)

    def fire(i, b):
        pltpu.async_copy(dst_hbm.at[pl.ds(base + i * KB, KB)], idxd[b], si[b])
        pltpu.async_copy(hs_hbm.at[idxs_all.at[pl.ds(i * KB, KB)]],
                         rows[b], sg[b])

    def finish(i, b):
        pltpu.make_async_copy(dst_hbm.at[pl.ds(0, KB)], idxd[b], si[b]).wait()
        pltpu.make_async_copy(hs_hbm.at[pl.ds(0, KB)], rows[b], sg[b]).wait()
        pltpu.sync_copy(rows[b], acc_sh.at[idxd[b]], add=True)

    for b in range(2):
        fire(b, b)

    def pair(i2, carry):
        for b in range(2):
            i = i2 * 2 + b
            finish(i, b)

            @pl.when(i + 2 < NFULL)
            def _():
                fire(i + 2, b)
        return carry

    lax.fori_loop(0, NFULL // 2, pair, 0)
    pltpu.sync_copy(dst_hbm.at[pl.ds(base + NFULL * KB, KTAIL)], idxt)
    pltpu.async_copy(hs_hbm.at[idxs_all.at[pl.ds(NFULL * KB, KTAIL)]],
                     rows0.at[pl.ds(0, KTAIL)], sg0).wait()
    pltpu.sync_copy(rows0.at[pl.ds(0, KTAIL)], acc_sh.at[idxt], add=True)
    plsc.subcore_barrier()

    for k in range(5):
        sl = pl.ds(sid * rpt + k * KB, KB)
        pltpu.sync_copy(acc_sh.at[sl], rows0)
        pltpu.sync_copy(rows0, out_hbm.at[cid, sl])


def _agg_call(src, dst, hs, width):
    zeros = jnp.zeros((128, width), jnp.float32)
    return pl.kernel(
        functools.partial(_agg_body, width),
        out_type=jax.ShapeDtypeStruct((NC, NP, width), jnp.float32),
        mesh=_mesh(),
        compiler_params=_SC_PARAMS,
        scratch_types=[
            pltpu.VMEM((EW,), jnp.int32),
            pltpu.VMEM((KB,), jnp.int32),
            pltpu.VMEM((KB,), jnp.int32),
            pltpu.VMEM((KTAIL,), jnp.int32),
            pltpu.VMEM((KB, width), jnp.float32),
            pltpu.VMEM((KB, width), jnp.float32),
            pltpu.VMEM_SHARED((NP, width), jnp.float32),
            pltpu.SemaphoreType.DMA,
            pltpu.SemaphoreType.DMA,
            pltpu.SemaphoreType.DMA,
            pltpu.SemaphoreType.DMA,
        ],
    )(src, dst, hs, zeros)


def _dec_body(src_hbm, dst_hbm, c_hbm, w_hbm, pb2_hbm, out_hbm,
              idxs_all, idxd_all, a0, a1, b0, b1, w_v, pb2_v, out_v,
              sa0, sa1, sb0, sb1):
    cid, sid, w = _wid()
    base = w * EW
    pltpu.sync_copy(src_hbm.at[pl.ds(base, EW)], idxs_all)
    pltpu.sync_copy(dst_hbm.at[pl.ds(base, EW)], idxd_all)
    pltpu.sync_copy(w_hbm, w_v)
    pltpu.sync_copy(pb2_hbm, pb2_v)
    pb2vec = pb2_v[...]
    lane = lax.iota(jnp.int32, L)
    abufs = (a0, a1)
    bbufs = (b0, b1)
    sas = (sa0, sa1)
    sbs = (sb0, sb1)

    def fire(i, b):
        pltpu.async_copy(c_hbm.at[idxs_all.at[pl.ds(i * KB, KB)]],
                         abufs[b], sas[b])
        pltpu.async_copy(c_hbm.at[idxd_all.at[pl.ds(i * KB, KB)]],
                         bbufs[b], sbs[b])

    def compute(a_v, b_v, g):
        res = jnp.zeros((L,), jnp.float32)
        for t in range(L):
            e = g * L + t
            acc = jnp.zeros((L,), jnp.float32)
            for q in range(HID // L):
                u = jnp.maximum(
                    a_v[e, pl.ds(q * L, L)]
                    + b_v[e, pl.ds(HID + q * L, L)], 0.0)
                acc = acc + u * w_v[pl.ds(q * L, L)]
            res = jnp.where(lane == t, jnp.sum(acc), res)
        out_v[pl.ds(g * L, L)] = res + pb2vec

    def finish(i, b):
        a_v, b_v = abufs[b], bbufs[b]
        pltpu.make_async_copy(c_hbm.at[pl.ds(0, KB)], a_v, sas[b]).wait()
        pltpu.make_async_copy(c_hbm.at[pl.ds(0, KB)], b_v, sbs[b]).wait()

        def grp(g, c2):
            compute(a_v, b_v, g)
            return c2

        lax.fori_loop(0, KB // L, grp, 0)
        pltpu.sync_copy(out_v, out_hbm.at[pl.ds(base + i * KB, KB)])

    for b in range(2):
        fire(b, b)

    def pair(i2, carry):
        for b in range(2):
            i = i2 * 2 + b
            finish(i, b)

            @pl.when(i + 2 < NFULL)
            def _():
                fire(i + 2, b)
        return carry

    lax.fori_loop(0, NFULL // 2, pair, 0)
    d1 = pltpu.async_copy(c_hbm.at[idxs_all.at[pl.ds(NFULL * KB, KTAIL)]],
                          a0.at[pl.ds(0, KTAIL)], sa0)
    d2 = pltpu.async_copy(c_hbm.at[idxd_all.at[pl.ds(NFULL * KB, KTAIL)]],
                          b0.at[pl.ds(0, KTAIL)], sb0)
    d1.wait()
    d2.wait()
    compute(a0, b0, 0)
    pltpu.sync_copy(out_v.at[pl.ds(0, KTAIL)],
                    out_hbm.at[pl.ds(base + NFULL * KB, KTAIL)])


def _dec_call(src, dst, ctab, w2, pb2):
    return pl.kernel(
        _dec_body,
        out_type=jax.ShapeDtypeStruct((E,), jnp.float32),
        mesh=_mesh(),
        compiler_params=_SC_PARAMS,
        scratch_types=[
            pltpu.VMEM((EW,), jnp.int32),
            pltpu.VMEM((EW,), jnp.int32),
            pltpu.VMEM((KB, IN_CH), jnp.float32),
            pltpu.VMEM((KB, IN_CH), jnp.float32),
            pltpu.VMEM((KB, IN_CH), jnp.float32),
            pltpu.VMEM((KB, IN_CH), jnp.float32),
            pltpu.VMEM((HID,), jnp.float32),
            pltpu.VMEM((L,), jnp.float32),
            pltpu.VMEM((KB,), jnp.float32),
            pltpu.SemaphoreType.DMA,
            pltpu.SemaphoreType.DMA,
            pltpu.SemaphoreType.DMA,
            pltpu.SemaphoreType.DMA,
        ],
    )(src, dst, ctab, w2, pb2)


_BR = 2000


def _tc1_body(x_ref, w1_ref, d0_ref, d1_ref, hs_ref, dinv_ref):
    deg = d0_ref[...] + d1_ref[...] + 1.0
    dinv = lax.rsqrt(deg)
    h = jnp.dot(x_ref[...], w1_ref[...], preferred_element_type=jnp.float32)
    hs_ref[...] = h * dinv
    dinv_ref[...] = dinv


def _tc1(x, W1, deg0, deg1):
    return pl.pallas_call(
        _tc1_body,
        grid=(N // _BR,),
        in_specs=[
            pl.BlockSpec((_BR, IN_CH), lambda i: (i, 0)),
            pl.BlockSpec((IN_CH, IN_CH), lambda i: (0, 0)),
            pl.BlockSpec((_BR, 1), lambda i: (i, 0)),
            pl.BlockSpec((_BR, 1), lambda i: (i, 0)),
        ],
        out_specs=[
            pl.BlockSpec((_BR, IN_CH), lambda i: (i, 0)),
            pl.BlockSpec((_BR, 1), lambda i: (i, 0)),
        ],
        out_shape=[
            jax.ShapeDtypeStruct((N, IN_CH), jnp.float32),
            jax.ShapeDtypeStruct((N, 1), jnp.float32),
        ],
    )(x, W1, deg0, deg1)


def _tc2_body(a0_ref, a1_ref, hs1_ref, dinv_ref, b1_ref, w2_ref, pw1r_ref,
              m_ref):
    dinv = dinv_ref[...]
    h = jnp.maximum(dinv * (a0_ref[...] + a1_ref[...] + hs1_ref[...]) + b1_ref[...], 0.0)
    hs2 = jnp.dot(h, w2_ref[...], preferred_element_type=jnp.float32) * dinv
    m_ref[...] = jnp.dot(hs2, pw1r_ref[...], preferred_element_type=jnp.float32)


def _tc2(a0, a1, hs1, dinv, b1, W2, PW1r):
    return pl.pallas_call(
        _tc2_body,
        grid=(N // _BR,),
        in_specs=[
            pl.BlockSpec((_BR, IN_CH), lambda i: (i, 0)),
            pl.BlockSpec((_BR, IN_CH), lambda i: (i, 0)),
            pl.BlockSpec((_BR, IN_CH), lambda i: (i, 0)),
            pl.BlockSpec((_BR, 1), lambda i: (i, 0)),
            pl.BlockSpec((1, IN_CH), lambda i: (0, 0)),
            pl.BlockSpec((IN_CH, HID), lambda i: (0, 0)),
            pl.BlockSpec((HID, IN_CH), lambda i: (0, 0)),
        ],
        out_specs=pl.BlockSpec((_BR, IN_CH), lambda i: (i, 0)),
        out_shape=jax.ShapeDtypeStruct((N, IN_CH), jnp.float32),
    )(a0, a1, hs1, dinv, b1, W2, PW1r)


def _tc3_body(s0_ref, s1_ref, m_ref, dinv_ref, b2_ref, pw1r_ref, pb1e_ref,
              c_ref):
    cab = jnp.dot(b2_ref[...], pw1r_ref[...],
                  preferred_element_type=jnp.float32) + pb1e_ref[...]
    c_ref[...] = dinv_ref[...] * (s0_ref[...] + s1_ref[...] + m_ref[...]) + cab


def _tc3(s0, s1, m, dinv, b2, PW1r, Pb1e):
    return pl.pallas_call(
        _tc3_body,
        grid=(N // _BR,),
        in_specs=[
            pl.BlockSpec((_BR, IN_CH), lambda i: (i, 0)),
            pl.BlockSpec((_BR, IN_CH), lambda i: (i, 0)),
            pl.BlockSpec((_BR, IN_CH), lambda i: (i, 0)),
            pl.BlockSpec((_BR, 1), lambda i: (i, 0)),
            pl.BlockSpec((1, HID), lambda i: (0, 0)),
            pl.BlockSpec((HID, IN_CH), lambda i: (0, 0)),
            pl.BlockSpec((1, IN_CH), lambda i: (0, 0)),
        ],
        out_specs=pl.BlockSpec((_BR, IN_CH), lambda i: (i, 0)),
        out_shape=jax.ShapeDtypeStruct((N, IN_CH), jnp.float32),
    )(s0, s1, m, dinv, b2, PW1r, Pb1e)


@jax.jit
def kernel(x, edge_index, W1, b1, W2, b2, PW1, Pb1, PW2, Pb2):
    src = edge_index[0].astype(jnp.int32)
    dst = edge_index[1].astype(jnp.int32)

    degs = _deg_call(dst)
    deg0 = degs[:N].reshape(N, 1)
    deg1 = degs[NP:NP + N].reshape(N, 1)

    PW1r = PW1.reshape(2, HID, HID).transpose(1, 0, 2).reshape(HID, 2 * HID)
    Pb1e = jnp.concatenate([Pb1, jnp.zeros_like(Pb1)]).reshape(1, 2 * HID)

    hs1, dinv = _tc1(x, W1, deg0, deg1)
    agg1 = _agg_call(src, dst, hs1, IN_CH)
    m = _tc2(agg1[0, :N], agg1[1, :N], hs1, dinv, b1.reshape(1, IN_CH), W2,
             PW1r)
    aggm = _agg_call(src, dst, m, IN_CH)
    ctab = _tc3(aggm[0, :N], aggm[1, :N], m, dinv, b2.reshape(1, HID),
                PW1r, Pb1e)
    out = _dec_call(src, dst, ctab, PW2.reshape(HID),
                    jnp.broadcast_to(Pb2, (L,)))
    return out


# speedup vs baseline: 23.3751x; 1.0013x over previous
import functools

import jax
import jax.numpy as jnp
from jax import lax
from jax.experimental import pallas as pl
from jax.experimental.pallas import tpu as pltpu
from jax.experimental.pallas import tpu_sc as plsc

N = 10000
E = 320000
IN_CH = 128
HID = 64
NC, NS, L = 2, 16, 16
NW = NC * NS
EW = E // NW
K = 80
KB = 128
NFULL = EW // KB
KTAIL = EW - NFULL * KB
NP = 10240
RPT = NP // NS

_mesh = functools.partial(
    plsc.VectorSubcoreMesh,
    core_axis_name="c", subcore_axis_name="s", num_cores=NC, num_subcores=NS,
)
_SC_PARAMS = pltpu.CompilerParams(needs_layout_passes=False)


def _wid():
    cid = lax.axis_index("c")
    sid = lax.axis_index("s")
    return cid, sid, sid * NC + cid


def _deg_body(dst_hbm, out_hbm, idx_all, deg_v, buf_v, spm, sem):
    cid, sid, w = _wid()
    base = w * EW
    ones = jnp.ones((L,), jnp.float32)

    def zero(j, carry):
        deg_v[pl.ds(j * L, L)] = jnp.zeros((L,), jnp.float32)
        return carry

    lax.fori_loop(0, NP // L, zero, 0)
    pltpu.sync_copy(dst_hbm.at[pl.ds(base, EW)], idx_all)

    def scat(j, c2):
        plsc.addupdate_scatter(deg_v, [idx_all[pl.ds(j * L, L)]], ones)
        return c2

    lax.fori_loop(0, EW // L, scat, 0)

    pltpu.sync_copy(deg_v, spm.at[pl.ds(sid * NP, NP)])
    plsc.subcore_barrier()
    lax.fori_loop(0, RPT // L, zero, 0)

    for r in range(NS):
        pltpu.sync_copy(spm.at[pl.ds(r * NP + sid * RPT, RPT)], buf_v)

        def add(j, carry):
            sl = pl.ds(j * L, L)
            deg_v[sl] = deg_v[sl] + buf_v[sl]
            return carry

        lax.fori_loop(0, RPT // L, add, 0)

    pltpu.sync_copy(deg_v.at[pl.ds(0, RPT)],
                    out_hbm.at[pl.ds(cid * NP + sid * RPT, RPT)])


def _deg_call(dst):
    return pl.kernel(
        _deg_body,
        out_type=jax.ShapeDtypeStruct((NC * NP,), jnp.float32),
        mesh=_mesh(),
        compiler_params=_SC_PARAMS,
        scratch_types=[
            pltpu.VMEM((EW,), jnp.int32),
            pltpu.VMEM((NP,), jnp.float32),
            pltpu.VMEM((RPT,), jnp.float32),
            pltpu.VMEM_SHARED((NS * NP,), jnp.float32),
            pltpu.SemaphoreType.DMA,
        ],
    )(dst)


def _agg_body(width, src_hbm, dst_hbm, hs_hbm, zeros_hbm, out_hbm,
              idxs_all, idxd0, idxd1, idxt, rows0, rows1, acc_sh,
              sg0, sg1, si0, si1):
    cid, sid, w = _wid()
    base = w * EW
    rpt = NP // NS
    for k in range(5):
        pltpu.sync_copy(zeros_hbm, acc_sh.at[pl.ds(sid * rpt + k * 128, 128)])
    pltpu.sync_copy(src_hbm.at[pl.ds(base, EW)], idxs_all)
    plsc.subcore_barrier()

    rows = (rows0, rows1)
    idxd = (idxd0, idxd1)
    sg = (sg0, sg1)
    si = (si0, si1)

    def fire(i, b):
        pltpu.async_copy(dst_hbm.at[pl.ds(base + i * KB, KB)], idxd[b], si[b])
        pltpu.async_copy(hs_hbm.at[idxs_all.at[pl.ds(i * KB, KB)]],
                         rows[b], sg[b])

    def finish(i, b):
        pltpu.make_async_copy(dst_hbm.at[pl.ds(0, KB)], idxd[b], si[b]).wait()
        pltpu.make_async_copy(hs_hbm.at[pl.ds(0, KB)], rows[b], sg[b]).wait()
        pltpu.sync_copy(rows[b], acc_sh.at[idxd[b]], add=True)

    for b in range(2):
        fire(b, b)

    def pair(i2, carry):
        for b in range(2):
            i = i2 * 2 + b
            finish(i, b)

            @pl.when(i + 2 < NFULL)
            def _():
                fire(i + 2, b)
        return carry

    lax.fori_loop(0, NFULL // 2, pair, 0)
    pltpu.sync_copy(dst_hbm.at[pl.ds(base + NFULL * KB, KTAIL)], idxt)
    pltpu.async_copy(hs_hbm.at[idxs_all.at[pl.ds(NFULL * KB, KTAIL)]],
                     rows0.at[pl.ds(0, KTAIL)], sg0).wait()
    pltpu.sync_copy(rows0.at[pl.ds(0, KTAIL)], acc_sh.at[idxt], add=True)
    plsc.subcore_barrier()

    for k in range(5):
        sl = pl.ds(sid * rpt + k * KB, KB)
        pltpu.sync_copy(acc_sh.at[sl], out_hbm.at[cid, sl])


def _agg_call(src, dst, hs, width):
    zeros = jnp.zeros((128, width), jnp.float32)
    return pl.kernel(
        functools.partial(_agg_body, width),
        out_type=jax.ShapeDtypeStruct((NC, NP, width), jnp.float32),
        mesh=_mesh(),
        compiler_params=_SC_PARAMS,
        scratch_types=[
            pltpu.VMEM((EW,), jnp.int32),
            pltpu.VMEM((KB,), jnp.int32),
            pltpu.VMEM((KB,), jnp.int32),
            pltpu.VMEM((KTAIL,), jnp.int32),
            pltpu.VMEM((KB, width), jnp.float32),
            pltpu.VMEM((KB, width), jnp.float32),
            pltpu.VMEM_SHARED((NP, width), jnp.float32),
            pltpu.SemaphoreType.DMA,
            pltpu.SemaphoreType.DMA,
            pltpu.SemaphoreType.DMA,
            pltpu.SemaphoreType.DMA,
        ],
    )(src, dst, hs, zeros)


def _dec_body(src_hbm, dst_hbm, c_hbm, w_hbm, pb2_hbm, out_hbm,
              idxs_all, idxd_all, a0, a1, b0, b1, w_v, pb2_v, o0, o1,
              sa0, sa1, sb0, sb1, so0, so1):
    cid, sid, w = _wid()
    base = w * EW
    pltpu.sync_copy(src_hbm.at[pl.ds(base, EW)], idxs_all)
    pltpu.sync_copy(dst_hbm.at[pl.ds(base, EW)], idxd_all)
    pltpu.sync_copy(w_hbm, w_v)
    pltpu.sync_copy(pb2_hbm, pb2_v)
    pb2vec = pb2_v[...]
    lane = lax.iota(jnp.int32, L)
    abufs = (a0, a1)
    bbufs = (b0, b1)
    obufs = (o0, o1)
    sas = (sa0, sa1)
    sbs = (sb0, sb1)
    sos = (so0, so1)

    def fire(i, b):
        pltpu.async_copy(c_hbm.at[idxs_all.at[pl.ds(i * KB, KB)]],
                         abufs[b], sas[b])
        pltpu.async_copy(c_hbm.at[idxd_all.at[pl.ds(i * KB, KB)]],
                         bbufs[b], sbs[b])

    def compute(a_v, b_v, out_v, g):
        res = jnp.zeros((L,), jnp.float32)
        for t in range(L):
            e = g * L + t
            acc = jnp.zeros((L,), jnp.float32)
            for q in range(HID // L):
                u = jnp.maximum(
                    a_v[e, pl.ds(q * L, L)]
                    + b_v[e, pl.ds(HID + q * L, L)], 0.0)
                acc = acc + u * w_v[pl.ds(q * L, L)]
            res = jnp.where(lane == t, jnp.sum(acc), res)
        out_v[pl.ds(g * L, L)] = res + pb2vec

    def finish(i, b):
        a_v, b_v, out_v = abufs[b], bbufs[b], obufs[b]
        pltpu.make_async_copy(c_hbm.at[pl.ds(0, KB)], a_v, sas[b]).wait()
        pltpu.make_async_copy(c_hbm.at[pl.ds(0, KB)], b_v, sbs[b]).wait()

        @pl.when(i >= 2)
        def _():
            pltpu.make_async_copy(
                out_v, out_hbm.at[pl.ds(base, KB)], sos[b]).wait()

        def grp(g, c2):
            compute(a_v, b_v, out_v, g)
            return c2

        lax.fori_loop(0, KB // L, grp, 0)
        pltpu.async_copy(out_v, out_hbm.at[pl.ds(base + i * KB, KB)], sos[b])

    for b in range(2):
        fire(b, b)

    def pair(i2, carry):
        for b in range(2):
            i = i2 * 2 + b
            finish(i, b)

            @pl.when(i + 2 < NFULL)
            def _():
                fire(i + 2, b)
        return carry

    lax.fori_loop(0, NFULL // 2, pair, 0)
    pltpu.make_async_copy(o0, out_hbm.at[pl.ds(base, KB)], so0).wait()
    pltpu.make_async_copy(o1, out_hbm.at[pl.ds(base, KB)], so1).wait()
    d1 = pltpu.async_copy(c_hbm.at[idxs_all.at[pl.ds(NFULL * KB, KTAIL)]],
                          a0.at[pl.ds(0, KTAIL)], sa0)
    d2 = pltpu.async_copy(c_hbm.at[idxd_all.at[pl.ds(NFULL * KB, KTAIL)]],
                          b0.at[pl.ds(0, KTAIL)], sb0)
    d1.wait()
    d2.wait()
    compute(a0, b0, o0, 0)
    pltpu.sync_copy(o0.at[pl.ds(0, KTAIL)],
                    out_hbm.at[pl.ds(base + NFULL * KB, KTAIL)])


def _dec_call(src, dst, ctab, w2, pb2):
    return pl.kernel(
        _dec_body,
        out_type=jax.ShapeDtypeStruct((E,), jnp.float32),
        mesh=_mesh(),
        compiler_params=_SC_PARAMS,
        scratch_types=[
            pltpu.VMEM((EW,), jnp.int32),
            pltpu.VMEM((EW,), jnp.int32),
            pltpu.VMEM((KB, IN_CH), jnp.float32),
            pltpu.VMEM((KB, IN_CH), jnp.float32),
            pltpu.VMEM((KB, IN_CH), jnp.float32),
            pltpu.VMEM((KB, IN_CH), jnp.float32),
            pltpu.VMEM((HID,), jnp.float32),
            pltpu.VMEM((L,), jnp.float32),
            pltpu.VMEM((KB,), jnp.float32),
            pltpu.VMEM((KB,), jnp.float32),
            pltpu.SemaphoreType.DMA,
            pltpu.SemaphoreType.DMA,
            pltpu.SemaphoreType.DMA,
            pltpu.SemaphoreType.DMA,
            pltpu.SemaphoreType.DMA,
            pltpu.SemaphoreType.DMA,
        ],
    )(src, dst, ctab, w2, pb2)


_BR = 2000


def _tc1_body(x_ref, w1_ref, d0_ref, d1_ref, hs_ref, dinv_ref):
    deg = d0_ref[...] + d1_ref[...] + 1.0
    dinv = lax.rsqrt(deg)
    h = jnp.dot(x_ref[...], w1_ref[...], preferred_element_type=jnp.float32)
    hs_ref[...] = h * dinv
    dinv_ref[...] = dinv


def _tc1(x, W1, deg0, deg1):
    return pl.pallas_call(
        _tc1_body,
        grid=(N // _BR,),
        in_specs=[
            pl.BlockSpec((_BR, IN_CH), lambda i: (i, 0)),
            pl.BlockSpec((IN_CH, IN_CH), lambda i: (0, 0)),
            pl.BlockSpec((_BR, 1), lambda i: (i, 0)),
            pl.BlockSpec((_BR, 1), lambda i: (i, 0)),
        ],
        out_specs=[
            pl.BlockSpec((_BR, IN_CH), lambda i: (i, 0)),
            pl.BlockSpec((_BR, 1), lambda i: (i, 0)),
        ],
        out_shape=[
            jax.ShapeDtypeStruct((N, IN_CH), jnp.float32),
            jax.ShapeDtypeStruct((N, 1), jnp.float32),
        ],
    )(x, W1, deg0, deg1)


def _tc2_body(a0_ref, a1_ref, hs1_ref, dinv_ref, b1_ref, w2_ref, pw1r_ref,
              m_ref):
    dinv = dinv_ref[...]
    h = jnp.maximum(dinv * (a0_ref[...] + a1_ref[...] + hs1_ref[...]) + b1_ref[...], 0.0)
    hs2 = jnp.dot(h, w2_ref[...], preferred_element_type=jnp.float32) * dinv
    m_ref[...] = jnp.dot(hs2, pw1r_ref[...], preferred_element_type=jnp.float32)


def _tc2(a0, a1, hs1, dinv, b1, W2, PW1r):
    return pl.pallas_call(
        _tc2_body,
        grid=(N // _BR,),
        in_specs=[
            pl.BlockSpec((_BR, IN_CH), lambda i: (i, 0)),
            pl.BlockSpec((_BR, IN_CH), lambda i: (i, 0)),
            pl.BlockSpec((_BR, IN_CH), lambda i: (i, 0)),
            pl.BlockSpec((_BR, 1), lambda i: (i, 0)),
            pl.BlockSpec((1, IN_CH), lambda i: (0, 0)),
            pl.BlockSpec((IN_CH, HID), lambda i: (0, 0)),
            pl.BlockSpec((HID, IN_CH), lambda i: (0, 0)),
        ],
        out_specs=pl.BlockSpec((_BR, IN_CH), lambda i: (i, 0)),
        out_shape=jax.ShapeDtypeStruct((N, IN_CH), jnp.float32),
    )(a0, a1, hs1, dinv, b1, W2, PW1r)


def _tc3_body(s0_ref, s1_ref, m_ref, dinv_ref, b2_ref, pw1r_ref, pb1e_ref,
              c_ref):
    cab = jnp.dot(b2_ref[...], pw1r_ref[...],
                  preferred_element_type=jnp.float32) + pb1e_ref[...]
    c_ref[...] = dinv_ref[...] * (s0_ref[...] + s1_ref[...] + m_ref[...]) + cab


def _tc3(s0, s1, m, dinv, b2, PW1r, Pb1e):
    return pl.pallas_call(
        _tc3_body,
        grid=(N // _BR,),
        in_specs=[
            pl.BlockSpec((_BR, IN_CH), lambda i: (i, 0)),
            pl.BlockSpec((_BR, IN_CH), lambda i: (i, 0)),
            pl.BlockSpec((_BR, IN_CH), lambda i: (i, 0)),
            pl.BlockSpec((_BR, 1), lambda i: (i, 0)),
            pl.BlockSpec((1, HID), lambda i: (0, 0)),
            pl.BlockSpec((HID, IN_CH), lambda i: (0, 0)),
            pl.BlockSpec((1, IN_CH), lambda i: (0, 0)),
        ],
        out_specs=pl.BlockSpec((_BR, IN_CH), lambda i: (i, 0)),
        out_shape=jax.ShapeDtypeStruct((N, IN_CH), jnp.float32),
    )(s0, s1, m, dinv, b2, PW1r, Pb1e)


@jax.jit
def kernel(x, edge_index, W1, b1, W2, b2, PW1, Pb1, PW2, Pb2):
    src = edge_index[0].astype(jnp.int32)
    dst = edge_index[1].astype(jnp.int32)

    degs = _deg_call(dst)
    deg0 = degs[:N].reshape(N, 1)
    deg1 = degs[NP:NP + N].reshape(N, 1)

    PW1r = PW1.reshape(2, HID, HID).transpose(1, 0, 2).reshape(HID, 2 * HID)
    Pb1e = jnp.concatenate([Pb1, jnp.zeros_like(Pb1)]).reshape(1, 2 * HID)

    hs1, dinv = _tc1(x, W1, deg0, deg1)
    agg1 = _agg_call(src, dst, hs1, IN_CH)
    m = _tc2(agg1[0, :N], agg1[1, :N], hs1, dinv, b1.reshape(1, IN_CH), W2,
             PW1r)
    aggm = _agg_call(src, dst, m, IN_CH)
    ctab = _tc3(aggm[0, :N], aggm[1, :N], m, dinv, b2.reshape(1, HID),
                PW1r, Pb1e)
    out = _dec_call(src, dst, ctab, PW2.reshape(HID),
                    jnp.broadcast_to(Pb2, (L,)))
    return out
